# baseline (speedup 1.0000x reference)
import functools

import jax
import jax.numpy as jnp
from jax import lax
from jax.experimental import pallas as pl
from jax.experimental.pallas import tpu as pltpu
from jax.experimental.pallas import tpu_sc as plsc

N = 10000
D_IN, D_HID, D_OUT = 128, 256, 128
E = 320000

NC, NS, LANES = 2, 16, 16
NW = NC * NS
N_ROWS = 10240
TRASH = 10000
ZROWS = N_ROWS // NS
CH = 128
E_PAD = 327680
EPT16 = E_PAD // NS
EPT32 = E_PAD // NW
NB = 2
DEPT = E // NW
DCH = 400
DNCHUNK = DEPT // DCH
HSLICE = N_ROWS // NS

BLK = 1000
GRID = N // BLK

_MESH = plsc.VectorSubcoreMesh(core_axis_name="c", subcore_axis_name="s")
_CP = pltpu.CompilerParams(needs_layout_passes=False)


@functools.partial(
    pl.kernel,
    out_type=jax.ShapeDtypeStruct((NC, N_ROWS), jnp.float32),
    mesh=_MESH,
    compiler_params=_CP,
    scratch_types=[
        pltpu.VMEM((DCH,), jnp.int32),
        pltpu.VMEM((N_ROWS,), jnp.float32),
        pltpu.VMEM((NS, HSLICE), jnp.float32),
        pltpu.VMEM((HSLICE,), jnp.float32),
        pltpu.VMEM_SHARED((NS, N_ROWS), jnp.float32),
    ],
)
def _degree_kernel(dst_hbm, out_hbm, dstbuf, hist, red, outbuf, stage):
    c = lax.axis_index("c")
    s = lax.axis_index("s")
    wid = s * NC + c
    zeros = jnp.zeros((LANES,), jnp.float32)
    for i in range(N_ROWS // LANES):
        hist[pl.ds(i * LANES, LANES)] = zeros
    ones = jnp.ones((LANES,), jnp.float32)

    def count_body(k, carry):
        base = wid * DEPT + k * DCH
        pltpu.sync_copy(dst_hbm.at[pl.ds(base, DCH)], dstbuf)
        for j in range(DCH // LANES):
            v = dstbuf[pl.ds(j * LANES, LANES)]
            plsc.addupdate_scatter(hist, [v], ones)
        return carry

    lax.fori_loop(0, DNCHUNK, count_body, 0)

    pltpu.sync_copy(hist, stage.at[s])
    plsc.subcore_barrier()
    pltpu.sync_copy(stage.at[:, pl.ds(s * HSLICE, HSLICE)], red)
    for j in range(HSLICE // LANES):
        acc = red[0, pl.ds(j * LANES, LANES)]
        for t in range(1, NS):
            acc = acc + red[t, pl.ds(j * LANES, LANES)]
        outbuf[pl.ds(j * LANES, LANES)] = acc
    pltpu.sync_copy(outbuf, out_hbm.at[c, pl.ds(s * HSLICE, HSLICE)])


def _make_scatter(split_edges_across_cores):
    D = 256
    H = D // 2
    nchunk = (EPT32 if split_edges_across_cores else EPT16) // CH

    @functools.partial(
        pl.kernel,
        out_type=jax.ShapeDtypeStruct((N_ROWS, D), jnp.float32),
        mesh=_MESH,
        compiler_params=_CP,
        scratch_types=(
            [pltpu.VMEM((CH,), jnp.int32)] * NB
            + [pltpu.VMEM((CH,), jnp.int32)] * NB
            + [pltpu.VMEM((CH, H), jnp.float32)] * NB
            + [pltpu.SemaphoreType.DMA] * NB
            + [
                pltpu.VMEM((8, H), jnp.float32),
                pltpu.VMEM_SHARED((N_ROWS, H), jnp.float32),
            ]
        ),
    )
    def _scatter_kernel(y_hbm, src_hbm, dst_hbm, out_hbm, *scratch):
        srcbuf = scratch[:NB]
        dstbuf = scratch[NB:2 * NB]
        rows = scratch[2 * NB:3 * NB]
        sem = scratch[3 * NB:4 * NB]
        zbuf, acc = scratch[4 * NB], scratch[4 * NB + 1]
        c = lax.axis_index("c")
        s = lax.axis_index("s")
        col = c * H
        zeros = jnp.zeros((LANES,), jnp.float32)
        for r in range(8):
            for j in range(H // LANES):
                zbuf[r, pl.ds(j * LANES, LANES)] = zeros

        def zero_body(k, carry):
            pltpu.sync_copy(zbuf, acc.at[pl.ds(s * ZROWS + k * 8, 8)])
            return carry

        lax.fori_loop(0, ZROWS // 8, zero_body, 0)
        plsc.subcore_barrier()
        if split_edges_across_cores:
            tile_base = c * (E_PAD // 2) + s * EPT32
        else:
            tile_base = s * EPT16

        def load_and_fire(k, b):
            base = tile_base + k * CH
            pltpu.sync_copy(src_hbm.at[pl.ds(base, CH)], srcbuf[b])
            pltpu.sync_copy(dst_hbm.at[pl.ds(base, CH)], dstbuf[b])
            pltpu.async_copy(y_hbm.at[srcbuf[b], pl.ds(col, H)],
                             rows[b], sem[b])

        def wait_and_add(b):
            pltpu.make_async_copy(y_hbm.at[srcbuf[b], pl.ds(col, H)],
                                  rows[b], sem[b]).wait()
            pltpu.sync_copy(rows[b], acc.at[dstbuf[b]], add=True)

        for b in range(NB):
            load_and_fire(b, b)

        def group_body(g, carry):
            for b in range(NB):
                wait_and_add(b)
                load_and_fire(g * NB + b + NB, b)
            return carry

        lax.fori_loop(0, nchunk // NB - 1, group_body, 0)
        for b in range(NB):
            wait_and_add(b)
        plsc.subcore_barrier()
        pltpu.sync_copy(acc.at[pl.ds(s * ZROWS, ZROWS)],
                        out_hbm.at[pl.ds(s * ZROWS, ZROWS), pl.ds(col, H)])

    return _scatter_kernel


_scatter_hid = _make_scatter(False)
_scatter_out = _make_scatter(True)


def _dis_block(degp_ref):
    deg = jnp.sum(degp_ref[...], axis=1, keepdims=True) + 1.0
    return lax.rsqrt(deg)


def _tc_a_body(degp_ref, x_ref, w_ref, y_ref):
    dis = _dis_block(degp_ref)
    xw = jnp.dot(x_ref[...], w_ref[...], preferred_element_type=jnp.float32)
    y_ref[...] = xw * dis


def _tc_b_body(degp_ref, agg_ref, y_ref, b_ref, w_ref, y2_ref):
    dis = _dis_block(degp_ref)
    h = jnp.maximum(dis * (agg_ref[...] + y_ref[...]) + b_ref[...], 0.0)
    y2 = jnp.dot(h, w_ref[...], preferred_element_type=jnp.float32) * dis
    y2_ref[...] = jnp.concatenate([y2, y2], axis=1)


def _tc_c_body(degp_ref, agg_ref, y_ref, b_ref, out_ref):
    dis = _dis_block(degp_ref)
    agg = agg_ref[...]
    y_self = y_ref[...]
    total = agg[:, :D_OUT] + agg[:, D_OUT:] + y_self[:, :D_OUT]
    out_ref[...] = dis * total + b_ref[...]


def _full(shape):
    return pl.BlockSpec(shape, lambda i: (0,) * len(shape))


def _rows(d):
    return pl.BlockSpec((BLK, d), lambda i: (i, 0))


_tc_a = pl.pallas_call(
    _tc_a_body,
    grid=(GRID,),
    in_specs=[_rows(NC), _rows(D_IN), _full((D_IN, D_HID))],
    out_specs=_rows(D_HID),
    out_shape=jax.ShapeDtypeStruct((N, D_HID), jnp.float32),
)

_tc_b = pl.pallas_call(
    _tc_b_body,
    grid=(GRID,),
    in_specs=[_rows(NC), _rows(D_HID), _rows(D_HID),
              _full((1, D_HID)), _full((D_HID, D_OUT))],
    out_specs=_rows(2 * D_OUT),
    out_shape=jax.ShapeDtypeStruct((N, 2 * D_OUT), jnp.float32),
)

_tc_c = pl.pallas_call(
    _tc_c_body,
    grid=(GRID,),
    in_specs=[_rows(NC), _rows(2 * D_OUT), _rows(2 * D_OUT), _full((1, D_OUT))],
    out_specs=_rows(D_OUT),
    out_shape=jax.ShapeDtypeStruct((N, D_OUT), jnp.float32),
)


def kernel(x, edge_index, W1, b1, W2, b2):
    src = edge_index[0].astype(jnp.int32)
    dst = edge_index[1].astype(jnp.int32)
    pad = E_PAD - E
    src_p = jnp.concatenate([src, jnp.zeros((pad,), jnp.int32)])
    trash = TRASH + (jnp.arange(pad, dtype=jnp.int32) % (N_ROWS - TRASH))
    dst_p = jnp.concatenate([dst, trash])

    deg_part = _degree_kernel(dst).T
    y1 = _tc_a(deg_part, x, W1)
    agg1 = _scatter_hid(y1, src_p, dst_p)
    y2 = _tc_b(deg_part, agg1, y1, b1.reshape(1, D_HID), W2)
    agg2 = _scatter_out(y2, src_p, dst_p)
    out = _tc_c(deg_part, agg2, y2, b2.reshape(1, D_OUT))
    return out


# speedup vs baseline: 1.0365x; 1.0365x over previous
import functools

import jax
import jax.numpy as jnp
from jax import lax
from jax.experimental import pallas as pl
from jax.experimental.pallas import tpu as pltpu
from jax.experimental.pallas import tpu_sc as plsc

N = 10000
D_IN, D_HID, D_OUT = 128, 256, 128
E = 320000

NC, NS, LANES = 2, 16, 16
NW = NC * NS
N_ROWS = 10240
TRASH = 10000
ZROWS = N_ROWS // NS
CH = 128
E_PAD = 327680
EPT16 = E_PAD // NS
EPT32 = E_PAD // NW
NB = 2
IB = 2048
CPB = IB // CH
DEPT = E // NW
DCH = 400
DNCHUNK = DEPT // DCH
HSLICE = N_ROWS // NS

BLK = 1000
GRID = N // BLK

_MESH = plsc.VectorSubcoreMesh(core_axis_name="c", subcore_axis_name="s")
_CP = pltpu.CompilerParams(needs_layout_passes=False)


@functools.partial(
    pl.kernel,
    out_type=jax.ShapeDtypeStruct((NC, N_ROWS), jnp.float32),
    mesh=_MESH,
    compiler_params=_CP,
    scratch_types=[
        pltpu.VMEM((DCH,), jnp.int32),
        pltpu.VMEM((N_ROWS,), jnp.float32),
        pltpu.VMEM((NS, HSLICE), jnp.float32),
        pltpu.VMEM((HSLICE,), jnp.float32),
        pltpu.VMEM_SHARED((NS, N_ROWS), jnp.float32),
    ],
)
def _degree_kernel(dst_hbm, out_hbm, dstbuf, hist, red, outbuf, stage):
    c = lax.axis_index("c")
    s = lax.axis_index("s")
    wid = s * NC + c
    zeros = jnp.zeros((LANES,), jnp.float32)
    for i in range(N_ROWS // LANES):
        hist[pl.ds(i * LANES, LANES)] = zeros
    ones = jnp.ones((LANES,), jnp.float32)

    def count_body(k, carry):
        base = wid * DEPT + k * DCH
        pltpu.sync_copy(dst_hbm.at[pl.ds(base, DCH)], dstbuf)
        for j in range(DCH // LANES):
            v = dstbuf[pl.ds(j * LANES, LANES)]
            plsc.addupdate_scatter(hist, [v], ones)
        return carry

    lax.fori_loop(0, DNCHUNK, count_body, 0)

    pltpu.sync_copy(hist, stage.at[s])
    plsc.subcore_barrier()
    pltpu.sync_copy(stage.at[:, pl.ds(s * HSLICE, HSLICE)], red)
    for j in range(HSLICE // LANES):
        acc = red[0, pl.ds(j * LANES, LANES)]
        for t in range(1, NS):
            acc = acc + red[t, pl.ds(j * LANES, LANES)]
        outbuf[pl.ds(j * LANES, LANES)] = acc
    pltpu.sync_copy(outbuf, out_hbm.at[c, pl.ds(s * HSLICE, HSLICE)])


def _make_scatter(split_edges_across_cores):
    D = 256
    H = D // 2
    ept = EPT32 if split_edges_across_cores else EPT16
    nblk = ept // IB

    @functools.partial(
        pl.kernel,
        out_type=jax.ShapeDtypeStruct((N_ROWS, D), jnp.float32),
        mesh=_MESH,
        compiler_params=_CP,
        scratch_types=(
            [
                pltpu.VMEM((IB,), jnp.int32),
                pltpu.VMEM((IB,), jnp.int32),
            ]
            + [pltpu.VMEM((CH, H), jnp.float32)] * NB
            + [pltpu.SemaphoreType.DMA] * NB
            + [
                pltpu.VMEM((8, H), jnp.float32),
                pltpu.VMEM_SHARED((N_ROWS, H), jnp.float32),
            ]
        ),
    )
    def _scatter_kernel(y_hbm, src_hbm, dst_hbm, out_hbm, *scratch):
        srcblk, dstblk = scratch[0], scratch[1]
        rows = scratch[2:2 + NB]
        sem = scratch[2 + NB:2 + 2 * NB]
        zbuf, acc = scratch[2 + 2 * NB], scratch[3 + 2 * NB]
        c = lax.axis_index("c")
        s = lax.axis_index("s")
        col = c * H
        zeros = jnp.zeros((LANES,), jnp.float32)
        for r in range(8):
            for j in range(H // LANES):
                zbuf[r, pl.ds(j * LANES, LANES)] = zeros

        def zero_body(k, carry):
            pltpu.sync_copy(zbuf, acc.at[pl.ds(s * ZROWS + k * 8, 8)])
            return carry

        lax.fori_loop(0, ZROWS // 8, zero_body, 0)
        if split_edges_across_cores:
            tile_base = c * (E_PAD // 2) + s * EPT32
        else:
            tile_base = s * EPT16
        plsc.subcore_barrier()

        def fire(k, b):
            pltpu.async_copy(
                y_hbm.at[srcblk.at[pl.ds(k * CH, CH)], pl.ds(col, H)],
                rows[b], sem[b])

        def wait_and_add(k, b):
            pltpu.make_async_copy(
                y_hbm.at[srcblk.at[pl.ds(k * CH, CH)], pl.ds(col, H)],
                rows[b], sem[b]).wait()
            pltpu.sync_copy(rows[b], acc.at[dstblk.at[pl.ds(k * CH, CH)]],
                            add=True)

        def block_body(blk, carry):
            base = tile_base + blk * IB
            pltpu.sync_copy(src_hbm.at[pl.ds(base, IB)], srcblk)
            pltpu.sync_copy(dst_hbm.at[pl.ds(base, IB)], dstblk)
            for b in range(NB):
                fire(b, b)
            for g in range(CPB - NB):
                wait_and_add(g, g % NB)
                fire(g + NB, (g + NB) % NB)
            for g in range(CPB - NB, CPB):
                wait_and_add(g, g % NB)
            return carry

        lax.fori_loop(0, nblk, block_body, 0)
        plsc.subcore_barrier()
        pltpu.sync_copy(acc.at[pl.ds(s * ZROWS, ZROWS)],
                        out_hbm.at[pl.ds(s * ZROWS, ZROWS), pl.ds(col, H)])

    return _scatter_kernel


_scatter_hid = _make_scatter(False)
_scatter_out = _make_scatter(True)


def _dis_block(degp_ref):
    deg = jnp.sum(degp_ref[...], axis=1, keepdims=True) + 1.0
    return lax.rsqrt(deg)


def _tc_a_body(degp_ref, x_ref, w_ref, y_ref):
    dis = _dis_block(degp_ref)
    xw = jnp.dot(x_ref[...], w_ref[...], preferred_element_type=jnp.float32)
    y_ref[...] = xw * dis


def _tc_b_body(degp_ref, agg_ref, y_ref, b_ref, w_ref, y2_ref):
    dis = _dis_block(degp_ref)
    h = jnp.maximum(dis * (agg_ref[...] + y_ref[...]) + b_ref[...], 0.0)
    y2 = jnp.dot(h, w_ref[...], preferred_element_type=jnp.float32) * dis
    y2_ref[...] = jnp.concatenate([y2, y2], axis=1)


def _tc_c_body(degp_ref, agg_ref, y_ref, b_ref, out_ref):
    dis = _dis_block(degp_ref)
    agg = agg_ref[...]
    y_self = y_ref[...]
    total = agg[:, :D_OUT] + agg[:, D_OUT:] + y_self[:, :D_OUT]
    out_ref[...] = dis * total + b_ref[...]


def _full(shape):
    return pl.BlockSpec(shape, lambda i: (0,) * len(shape))


def _rows(d):
    return pl.BlockSpec((BLK, d), lambda i: (i, 0))


_tc_a = pl.pallas_call(
    _tc_a_body,
    grid=(GRID,),
    in_specs=[_rows(NC), _rows(D_IN), _full((D_IN, D_HID))],
    out_specs=_rows(D_HID),
    out_shape=jax.ShapeDtypeStruct((N, D_HID), jnp.float32),
)

_tc_b = pl.pallas_call(
    _tc_b_body,
    grid=(GRID,),
    in_specs=[_rows(NC), _rows(D_HID), _rows(D_HID),
              _full((1, D_HID)), _full((D_HID, D_OUT))],
    out_specs=_rows(2 * D_OUT),
    out_shape=jax.ShapeDtypeStruct((N, 2 * D_OUT), jnp.float32),
)

_tc_c = pl.pallas_call(
    _tc_c_body,
    grid=(GRID,),
    in_specs=[_rows(NC), _rows(2 * D_OUT), _rows(2 * D_OUT), _full((1, D_OUT))],
    out_specs=_rows(D_OUT),
    out_shape=jax.ShapeDtypeStruct((N, D_OUT), jnp.float32),
)


def kernel(x, edge_index, W1, b1, W2, b2):
    src = edge_index[0].astype(jnp.int32)
    dst = edge_index[1].astype(jnp.int32)
    pad = E_PAD - E
    src_p = jnp.concatenate([src, jnp.zeros((pad,), jnp.int32)])
    trash = TRASH + (jnp.arange(pad, dtype=jnp.int32) % (N_ROWS - TRASH))
    dst_p = jnp.concatenate([dst, trash])

    deg_part = _degree_kernel(dst).T
    y1 = _tc_a(deg_part, x, W1)
    agg1 = _scatter_hid(y1, src_p, dst_p)
    y2 = _tc_b(deg_part, agg1, y1, b1.reshape(1, D_HID), W2)
    agg2 = _scatter_out(y2, src_p, dst_p)
    out = _tc_c(deg_part, agg2, y2, b2.reshape(1, D_OUT))
    return out


# speedup vs baseline: 2.8866x; 2.7850x over previous
import functools

import jax
import jax.numpy as jnp
from jax import lax
from jax.experimental import pallas as pl
from jax.experimental.pallas import tpu as pltpu
from jax.experimental.pallas import tpu_sc as plsc

N = 10000
D_IN, D_HID, D_OUT = 128, 256, 128
E = 320000

NC, NS, LANES = 2, 16, 16
NW = NC * NS
N_ROWS = 10240
TRASH = 10000
ZROWS = N_ROWS // NS
CH = 128
E_PAD = 327680
EPT16 = E_PAD // NS
EPT32 = E_PAD // NW
NB = 2
IB = 2048
CPB = IB // CH
DEPT = E // NW
DCH = 400
DNCHUNK = DEPT // DCH
HSLICE = N_ROWS // NS

BLK = 1000
GRID = N // BLK

_MESH = plsc.VectorSubcoreMesh(core_axis_name="c", subcore_axis_name="s")
_CP = pltpu.CompilerParams(needs_layout_passes=False)


@functools.partial(
    pl.kernel,
    out_type=jax.ShapeDtypeStruct((NC, N_ROWS), jnp.float32),
    mesh=_MESH,
    compiler_params=_CP,
    scratch_types=[
        pltpu.VMEM((DCH,), jnp.int32),
        pltpu.VMEM((N_ROWS,), jnp.float32),
        pltpu.VMEM((NS, HSLICE), jnp.float32),
        pltpu.VMEM((HSLICE,), jnp.float32),
        pltpu.VMEM_SHARED((NS, N_ROWS), jnp.float32),
    ],
)
def _degree_kernel(dst_hbm, out_hbm, dstbuf, hist, red, outbuf, stage):
    c = lax.axis_index("c")
    s = lax.axis_index("s")
    wid = s * NC + c
    zeros = jnp.zeros((LANES,), jnp.float32)
    for i in range(N_ROWS // LANES):
        hist[pl.ds(i * LANES, LANES)] = zeros
    ones = jnp.ones((LANES,), jnp.float32)

    def count_body(k, carry):
        base = wid * DEPT + k * DCH
        pltpu.sync_copy(dst_hbm.at[pl.ds(base, DCH)], dstbuf)
        for j in range(DCH // LANES):
            v = dstbuf[pl.ds(j * LANES, LANES)]
            plsc.addupdate_scatter(hist, [v], ones)
        return carry

    lax.fori_loop(0, DNCHUNK, count_body, 0)

    pltpu.sync_copy(hist, stage.at[s])
    plsc.subcore_barrier()
    pltpu.sync_copy(stage.at[:, pl.ds(s * HSLICE, HSLICE)], red)
    for j in range(HSLICE // LANES):
        acc = red[0, pl.ds(j * LANES, LANES)]
        for t in range(1, NS):
            acc = acc + red[t, pl.ds(j * LANES, LANES)]
        outbuf[pl.ds(j * LANES, LANES)] = acc
    pltpu.sync_copy(outbuf, out_hbm.at[c, pl.ds(s * HSLICE, HSLICE)])


def _make_scatter(split_edges_across_cores):
    D = 256
    H = D // 2
    ept = EPT32 if split_edges_across_cores else EPT16
    nblk = ept // IB

    @functools.partial(
        pl.kernel,
        out_type=jax.ShapeDtypeStruct((N_ROWS, D), jnp.float32),
        mesh=_MESH,
        compiler_params=_CP,
        scratch_types=(
            [
                pltpu.VMEM((IB,), jnp.int32),
                pltpu.VMEM((IB,), jnp.int32),
            ]
            + [pltpu.VMEM((CH, H), jnp.float32)] * NB
            + [pltpu.SemaphoreType.DMA] * NB
            + [
                pltpu.VMEM((8, H), jnp.float32),
                pltpu.VMEM_SHARED((N_ROWS, H), jnp.float32),
            ]
        ),
    )
    def _scatter_kernel(y_hbm, src_hbm, dst_hbm, out_hbm, *scratch):
        srcblk, dstblk = scratch[0], scratch[1]
        rows = scratch[2:2 + NB]
        sem = scratch[2 + NB:2 + 2 * NB]
        zbuf, acc = scratch[2 + 2 * NB], scratch[3 + 2 * NB]
        c = lax.axis_index("c")
        s = lax.axis_index("s")
        col = c * H
        zeros = jnp.zeros((LANES,), jnp.float32)
        for r in range(8):
            for j in range(H // LANES):
                zbuf[r, pl.ds(j * LANES, LANES)] = zeros

        def zero_body(k, carry):
            pltpu.sync_copy(zbuf, acc.at[pl.ds(s * ZROWS + k * 8, 8)])
            return carry

        lax.fori_loop(0, ZROWS // 8, zero_body, 0)
        if split_edges_across_cores:
            tile_base = c * (E_PAD // 2) + s * EPT32
        else:
            tile_base = s * EPT16
        plsc.subcore_barrier()

        def fire(k, b):
            pltpu.async_copy(
                y_hbm.at[srcblk.at[pl.ds(k * CH, CH)], pl.ds(col, H)],
                rows[b], sem[b])

        def wait_and_add(k, b):
            pltpu.make_async_copy(
                y_hbm.at[srcblk.at[pl.ds(k * CH, CH)], pl.ds(col, H)],
                rows[b], sem[b]).wait()
            pltpu.sync_copy(rows[b], acc.at[dstblk.at[pl.ds(k * CH, CH)]],
                            add=True)

        def block_body(blk, carry):
            base = tile_base + blk * IB
            pltpu.sync_copy(src_hbm.at[pl.ds(base, IB)], srcblk)
            pltpu.sync_copy(dst_hbm.at[pl.ds(base, IB)], dstblk)
            for b in range(NB):
                fire(b, b)
            for g in range(CPB - NB):
                wait_and_add(g, g % NB)
                fire(g + NB, (g + NB) % NB)
            for g in range(CPB - NB, CPB):
                wait_and_add(g, g % NB)
            return carry

        lax.fori_loop(0, nblk, block_body, 0)
        plsc.subcore_barrier()
        pltpu.sync_copy(acc.at[pl.ds(s * ZROWS, ZROWS)],
                        out_hbm.at[pl.ds(s * ZROWS, ZROWS), pl.ds(col, H)])

    return _scatter_kernel


_scatter_hid = _make_scatter(False)
_scatter_out = _make_scatter(True)


def _dis_block(degp_ref):
    deg = jnp.sum(degp_ref[...], axis=1, keepdims=True) + 1.0
    return lax.rsqrt(deg)


def _tc_a_body(degp_ref, x_ref, w_ref, y_ref):
    dis = _dis_block(degp_ref)
    xw = jnp.dot(x_ref[...], w_ref[...], preferred_element_type=jnp.float32)
    y_ref[...] = xw * dis


def _tc_b_body(degp_ref, agg_ref, y_ref, b_ref, w_ref, y2_ref):
    dis = _dis_block(degp_ref)
    h = jnp.maximum(dis * (agg_ref[...] + y_ref[...]) + b_ref[...], 0.0)
    y2 = jnp.dot(h, w_ref[...], preferred_element_type=jnp.float32) * dis
    y2_ref[...] = jnp.concatenate([y2, y2], axis=1)


def _tc_c_body(degp_ref, agg_ref, y_ref, b_ref, out_ref):
    dis = _dis_block(degp_ref)
    agg = agg_ref[...]
    y_self = y_ref[...]
    total = agg[:, :D_OUT] + agg[:, D_OUT:] + y_self[:, :D_OUT]
    out_ref[...] = dis * total + b_ref[...]


def _full(shape):
    return pl.BlockSpec(shape, lambda i: (0,) * len(shape))


def _rows(d):
    return pl.BlockSpec((BLK, d), lambda i: (i, 0))


_tc_a = pl.pallas_call(
    _tc_a_body,
    grid=(GRID,),
    in_specs=[_rows(NC), _rows(D_IN), _full((D_IN, D_HID))],
    out_specs=_rows(D_HID),
    out_shape=jax.ShapeDtypeStruct((N, D_HID), jnp.float32),
)

_tc_b = pl.pallas_call(
    _tc_b_body,
    grid=(GRID,),
    in_specs=[_rows(NC), _rows(D_HID), _rows(D_HID),
              _full((1, D_HID)), _full((D_HID, D_OUT))],
    out_specs=_rows(2 * D_OUT),
    out_shape=jax.ShapeDtypeStruct((N, 2 * D_OUT), jnp.float32),
)

_tc_c = pl.pallas_call(
    _tc_c_body,
    grid=(GRID,),
    in_specs=[_rows(NC), _rows(2 * D_OUT), _rows(2 * D_OUT), _full((1, D_OUT))],
    out_specs=_rows(D_OUT),
    out_shape=jax.ShapeDtypeStruct((N, D_OUT), jnp.float32),
)


def kernel(x, edge_index, W1, b1, W2, b2):
    src = edge_index[0].astype(jnp.int32)
    dst = edge_index[1].astype(jnp.int32)
    pad = E_PAD - E
    src_p = jnp.concatenate([src, jnp.arange(pad, dtype=jnp.int32) % N])
    trash = TRASH + (jnp.arange(pad, dtype=jnp.int32) % (N_ROWS - TRASH))
    dst_p = jnp.concatenate([dst, trash])

    deg_part = _degree_kernel(dst).T
    y1 = _tc_a(deg_part, x, W1)
    agg1 = _scatter_hid(y1, src_p, dst_p)
    y2 = _tc_b(deg_part, agg1, y1, b1.reshape(1, D_HID), W2)
    agg2 = _scatter_out(y2, src_p, dst_p)
    out = _tc_c(deg_part, agg2, y2, b2.reshape(1, D_OUT))
    return out
